# baseline (speedup 1.0000x reference)
import functools
import math

import jax
import jax.numpy as jnp
from jax import lax
from jax.experimental import pallas as pl
from jax.experimental.pallas import tpu as pltpu
from jax.experimental.pallas import tpu_sc as plsc

_D = 64
_DP = 128
_SCALE = math.sqrt(_D)
_NC = 2
_NS = 16
_L = 16
_NW = _NC * _NS
_CHUNK = 80
_T = 20
_BC = _CHUNK // _T


_V = 1000000
_VB = 128
_NBF = _V // _VB
_VT = _V - _NBF * _VB
_BPT = (_NBF + _NW - 1) // _NW


@functools.lru_cache(maxsize=None)
def _build_transpose():
    mesh = plsc.VectorSubcoreMesh(core_axis_name="c", subcore_axis_name="s")

    @functools.partial(
        pl.kernel,
        mesh=mesh,
        out_type=jax.ShapeDtypeStruct((_V // 2, _DP), jnp.float32),
        compiler_params=pltpu.CompilerParams(
            use_tc_tiling_on_sc=True, needs_layout_passes=False),
        scratch_types=[
            pltpu.VMEM((_D, _VB), jnp.float32),
            pltpu.VMEM((_D, _VB), jnp.float32),
            pltpu.VMEM((_VB // 2, _DP), jnp.float32),
            pltpu.VMEM((_VB // 2, _DP), jnp.float32),
            pltpu.SemaphoreType.DMA,
            pltpu.SemaphoreType.DMA,
            pltpu.SemaphoreType.DMA,
            pltpu.SemaphoreType.DMA,
        ],
    )
    def tr(tabt_hbm, tail_hbm, out_hbm, ina, inb, outa, outb,
           gsa, gsb, osa, osb):
        wid = lax.axis_index("s") * _NC + lax.axis_index("c")
        blk0 = wid * _BPT
        ins = (ina, inb)
        outs = (outa, outb)
        gsems = (gsa, gsb)
        osems = (osa, osb)
        lanes = lax.iota(jnp.int32, _L)

        def fire_in(blk, j):
            @pl.when(blk < _NBF)
            def _():
                v0 = pl.multiple_of(blk * _VB, _VB)
                pltpu.async_copy(
                    tabt_hbm.at[:, pl.ds(v0, _VB)], ins[j], gsems[j])

        def compute(j, width):
            rowsel = [lanes + d0 for d0 in range(0, _D, _L)]
            for p in range(width // 2):
                for h in (0, 1):
                    col = lanes * 0 + (2 * p + h)
                    for q in range(_D // _L):
                        vals = plsc.load_gather(ins[j], [rowsel[q], col])
                        outs[j][p, pl.ds(h * _D + q * _L, _L)] = vals

        fire_in(blk0, 0)
        fire_in(blk0 + 1, 1)

        def body(k, carry):
            for j in range(2):
                i = 2 * k + j
                blk = blk0 + i

                @pl.when((i < _BPT) & (blk < _NBF))
                def _():
                    pltpu.make_async_copy(
                        tabt_hbm.at[:, pl.ds(0, _VB)], ins[j], gsems[j]
                    ).wait()

                    @pl.when(k > 0)
                    def _():
                        pltpu.make_async_copy(
                            outs[j], out_hbm.at[pl.ds(0, _VB // 2)],
                            osems[j]).wait()

                    compute(j, _VB)
                    pltpu.async_copy(
                        outs[j],
                        out_hbm.at[pl.ds(blk * (_VB // 2), _VB // 2)],
                        osems[j])

                    @pl.when(i + 2 < _BPT)
                    def _():
                        fire_in(blk + 2, j)
            return carry

        lax.fori_loop(0, (_BPT + 1) // 2, body, 0)
        for j in range(2):
            @pl.when(blk0 + j < _NBF)
            def _():
                pltpu.make_async_copy(
                    outs[j], out_hbm.at[pl.ds(0, _VB // 2)], osems[j]).wait()

        @pl.when(wid == _NW - 1)
        def _():
            pltpu.sync_copy(tail_hbm, outa.at[pl.ds(0, _VT // 2)])
            pltpu.sync_copy(
                outa.at[pl.ds(0, _VT // 2)],
                out_hbm.at[pl.ds(_NBF * (_VB // 2), _VT // 2)])

    return tr


@functools.lru_cache(maxsize=None)
def _build(BATCH: int):
    B = BATCH * _T
    BPW = B // _NW
    NCH = BPW // _CHUNK
    BW = BATCH // _NW

    mesh = plsc.VectorSubcoreMesh(core_axis_name="c", subcore_axis_name="s")

    @functools.partial(
        pl.kernel,
        mesh=mesh,
        out_type=jax.ShapeDtypeStruct((BATCH, _T, _D), jnp.float32),
        compiler_params=pltpu.CompilerParams(use_tc_tiling_on_sc=True),
        scratch_types=[
            pltpu.VMEM((NCH, _CHUNK), jnp.int32),
            pltpu.VMEM((NCH, _CHUNK), jnp.float32),
            pltpu.VMEM((_CHUNK, _DP), jnp.float32),
            pltpu.VMEM((_CHUNK, _DP), jnp.float32),
            pltpu.VMEM((_BC, _T, _D), jnp.float32),
            pltpu.VMEM((_BC, _T, _D), jnp.float32),
            pltpu.SemaphoreType.DMA,
            pltpu.SemaphoreType.DMA,
            pltpu.SemaphoreType.DMA,
            pltpu.SemaphoreType.DMA,
        ],
    )
    def emb(idx_hbm, par_hbm, tab_hbm, out_hbm, idx_v, par_v, ina, inb,
            outa, outb, gsa, gsb, osa, osb):
        wid = lax.axis_index("s") * _NC + lax.axis_index("c")
        bbase = wid * BW
        pltpu.sync_copy(idx_hbm.at[wid], idx_v)
        pltpu.sync_copy(par_hbm.at[wid], par_v)

        ins = (ina, inb)
        outs = (outa, outb)
        gsems = (gsa, gsb)
        osems = (osa, osb)

        def fire_gather(c, j):
            pltpu.async_copy(tab_hbm.at[idx_v.at[c]], ins[j], gsems[j])

        def compute(c, j):
            lanes = lax.iota(jnp.int32, _L)
            for r0 in range(0, _CHUNK, _L):
                pv = par_v[c, pl.ds(r0, _L)]
                for l in range(_L):
                    r = r0 + l
                    bl, tt = divmod(r, _T)
                    hi_w = lax.gather(
                        pv, (lanes * 0 + l)[:, None],
                        lax.GatherDimensionNumbers(
                            offset_dims=(), collapsed_slice_dims=(0,),
                            start_index_map=(0,)),
                        (1,),
                        mode=lax.GatherScatterMode.PROMISE_IN_BOUNDS)
                    lo_w = _SCALE - hi_w
                    for jj in range(_D // _L):
                        sl = pl.ds(jj * _L, _L)
                        lo = ins[j][r, sl]
                        hi = ins[j][r, pl.ds(_D + jj * _L, _L)]
                        outs[j][bl, tt, sl] = lo * lo_w + hi * hi_w

        fire_gather(0, 0)
        fire_gather(1, 1)

        def body(k, carry):
            for j in range(2):
                c = 2 * k + j
                pltpu.make_async_copy(
                    tab_hbm.at[idx_v.at[0]], ins[j], gsems[j]).wait()
                @pl.when(k > 0)
                def _():
                    pltpu.make_async_copy(
                        outs[j], out_hbm.at[pl.ds(bbase, _BC)], osems[j]
                    ).wait()
                compute(c, j)
                pltpu.async_copy(
                    outs[j],
                    out_hbm.at[pl.ds(bbase + c * _BC, _BC)],
                    osems[j],
                )
                @pl.when(c + 2 < NCH)
                def _():
                    fire_gather(c + 2, j)
            return carry

        lax.fori_loop(0, NCH // 2, body, 0)
        for j in range(2):
            pltpu.make_async_copy(
                outs[j], out_hbm.at[pl.ds(bbase, _BC)], osems[j]).wait()

    return emb


def kernel(token, lookup_table):
    BATCH = token.shape[0]
    B = BATCH * _T
    tabf = lookup_table.astype(jnp.float32)
    tab2 = _build_transpose()(
        tabf.T, tabf[_NBF * _VB:].reshape(_VT // 2, _DP))
    tok = token.reshape(-1).astype(jnp.int32)
    idx = (tok >> 1).reshape(_NW, (B // _NW) // _CHUNK, _CHUNK)
    par = ((tok & 1).astype(jnp.float32) * _SCALE).reshape(
        _NW, (B // _NW) // _CHUNK, _CHUNK)
    return _build(BATCH)(idx, par, tab2)


# speedup vs baseline: 2.9019x; 2.9019x over previous
import functools
import math

import jax
import jax.numpy as jnp
from jax import lax
from jax.experimental import pallas as pl
from jax.experimental.pallas import tpu as pltpu
from jax.experimental.pallas import tpu_sc as plsc

_D = 64
_DP = 128
_SCALE = math.sqrt(_D)
_NC = 2
_NS = 16
_L = 16
_NW = _NC * _NS
_CHUNK = 80
_T = 20
_BC = _CHUNK // _T

_V = 1000000
_VBLK = 4096


@functools.lru_cache(maxsize=None)
def _build_transpose():
    grid = (_V + _VBLK - 1) // _VBLK

    def tr(tabt_ref, out_ref):
        out_ref[:, 0:_D] = tabt_ref[...].T

    return pl.pallas_call(
        tr,
        grid=(grid,),
        in_specs=[pl.BlockSpec((_D, _VBLK), lambda i: (0, i))],
        out_specs=pl.BlockSpec((_VBLK, _DP), lambda i: (i, 0)),
        out_shape=jax.ShapeDtypeStruct((_V, _DP), jnp.float32),
    )


@functools.lru_cache(maxsize=None)
def _build(BATCH: int):
    B = BATCH * _T
    BPW = B // _NW
    NCH = BPW // _CHUNK
    BW = BATCH // _NW

    mesh = plsc.VectorSubcoreMesh(core_axis_name="c", subcore_axis_name="s")

    @functools.partial(
        pl.kernel,
        mesh=mesh,
        out_type=jax.ShapeDtypeStruct((BATCH, _T, _D), jnp.float32),
        compiler_params=pltpu.CompilerParams(use_tc_tiling_on_sc=True),
        scratch_types=[
            pltpu.VMEM((NCH, _CHUNK), jnp.int32),
            pltpu.VMEM((_CHUNK, _DP), jnp.float32),
            pltpu.VMEM((_CHUNK, _DP), jnp.float32),
            pltpu.VMEM((_BC, _T, _D), jnp.float32),
            pltpu.VMEM((_BC, _T, _D), jnp.float32),
            pltpu.SemaphoreType.DMA,
            pltpu.SemaphoreType.DMA,
            pltpu.SemaphoreType.DMA,
            pltpu.SemaphoreType.DMA,
        ],
    )
    def emb(idx_hbm, tab_hbm, out_hbm, idx_v, ina, inb, outa, outb,
            gsa, gsb, osa, osb):
        wid = lax.axis_index("s") * _NC + lax.axis_index("c")
        bbase = wid * BW
        pltpu.sync_copy(idx_hbm.at[wid], idx_v)

        ins = (ina, inb)
        outs = (outa, outb)
        gsems = (gsa, gsb)
        osems = (osa, osb)

        def fire_gather(c, j):
            pltpu.async_copy(tab_hbm.at[idx_v.at[c]], ins[j], gsems[j])

        def compute(j):
            for r in range(_CHUNK):
                bl, tt = divmod(r, _T)
                for jj in range(_D // _L):
                    sl = pl.ds(jj * _L, _L)
                    outs[j][bl, tt, sl] = ins[j][r, sl] * _SCALE

        fire_gather(0, 0)
        fire_gather(1, 1)

        def body(k, carry):
            for j in range(2):
                c = 2 * k + j
                pltpu.make_async_copy(
                    tab_hbm.at[idx_v.at[0]], ins[j], gsems[j]).wait()
                @pl.when(k > 0)
                def _():
                    pltpu.make_async_copy(
                        outs[j], out_hbm.at[pl.ds(bbase, _BC)], osems[j]
                    ).wait()
                compute(j)
                pltpu.async_copy(
                    outs[j],
                    out_hbm.at[pl.ds(bbase + c * _BC, _BC)],
                    osems[j],
                )
                @pl.when(c + 2 < NCH)
                def _():
                    fire_gather(c + 2, j)
            return carry

        lax.fori_loop(0, NCH // 2, body, 0)
        for j in range(2):
            pltpu.make_async_copy(
                outs[j], out_hbm.at[pl.ds(bbase, _BC)], osems[j]).wait()

    return emb


def kernel(token, lookup_table):
    BATCH = token.shape[0]
    B = BATCH * _T
    tab128 = _build_transpose()(lookup_table.astype(jnp.float32).T)
    idx = token.reshape(-1).astype(jnp.int32)
    idx = idx.reshape(_NW, (B // _NW) // _CHUNK, _CHUNK)
    return _build(BATCH)(idx, tab128)


# speedup vs baseline: 3.2757x; 1.1288x over previous
import functools
import math

import jax
import jax.numpy as jnp
from jax import lax
from jax.experimental import pallas as pl
from jax.experimental.pallas import tpu as pltpu
from jax.experimental.pallas import tpu_sc as plsc

_D = 64
_DP = 128
_SCALE = math.sqrt(_D)
_NC = 2
_NS = 16
_L = 16
_NW = _NC * _NS
_CHUNK = 80
_T = 20
_BC = _CHUNK // _T

_V = 1000000
_VBLK = 8192


@functools.lru_cache(maxsize=None)
def _build_transpose():
    grid = (_V + _VBLK - 1) // _VBLK

    def tr(tabt_ref, out_ref):
        out_ref[:, 0:_D] = tabt_ref[...].T

    return pl.pallas_call(
        tr,
        grid=(grid,),
        in_specs=[pl.BlockSpec((_D, _VBLK), lambda i: (0, i))],
        out_specs=pl.BlockSpec((_VBLK, _DP), lambda i: (i, 0)),
        out_shape=jax.ShapeDtypeStruct((_V, _DP), jnp.float32),
    )


@functools.lru_cache(maxsize=None)
def _build(BATCH: int):
    B = BATCH * _T
    BPW = B // _NW
    NCH = BPW // _CHUNK
    BW = BATCH // _NW

    mesh = plsc.VectorSubcoreMesh(core_axis_name="c", subcore_axis_name="s")

    @functools.partial(
        pl.kernel,
        mesh=mesh,
        out_type=jax.ShapeDtypeStruct((BATCH, _T, _D), jnp.float32),
        compiler_params=pltpu.CompilerParams(use_tc_tiling_on_sc=True),
        scratch_types=[
            pltpu.VMEM((NCH, _CHUNK), jnp.int32),
            pltpu.VMEM((_CHUNK, _DP), jnp.float32),
            pltpu.VMEM((_CHUNK, _DP), jnp.float32),
            pltpu.VMEM((_BC, _T, _D), jnp.float32),
            pltpu.VMEM((_BC, _T, _D), jnp.float32),
            pltpu.SemaphoreType.DMA,
            pltpu.SemaphoreType.DMA,
            pltpu.SemaphoreType.DMA,
            pltpu.SemaphoreType.DMA,
        ],
    )
    def emb(idx_hbm, tab_hbm, out_hbm, idx_v, ina, inb, outa, outb,
            gsa, gsb, osa, osb):
        wid = lax.axis_index("s") * _NC + lax.axis_index("c")
        bbase = wid * BW
        pltpu.sync_copy(idx_hbm.at[wid], idx_v)

        ins = (ina, inb)
        outs = (outa, outb)
        gsems = (gsa, gsb)
        osems = (osa, osb)

        def fire_gather(c, j):
            pltpu.async_copy(tab_hbm.at[idx_v.at[c]], ins[j], gsems[j])

        def compute(j):
            for r in range(_CHUNK):
                bl, tt = divmod(r, _T)
                for jj in range(_D // _L):
                    sl = pl.ds(jj * _L, _L)
                    outs[j][bl, tt, sl] = ins[j][r, sl] * _SCALE

        fire_gather(0, 0)
        fire_gather(1, 1)

        def body(k, carry):
            for j in range(2):
                c = 2 * k + j
                pltpu.make_async_copy(
                    tab_hbm.at[idx_v.at[0]], ins[j], gsems[j]).wait()
                @pl.when(k > 0)
                def _():
                    pltpu.make_async_copy(
                        outs[j], out_hbm.at[pl.ds(bbase, _BC)], osems[j]
                    ).wait()
                compute(j)
                pltpu.async_copy(
                    outs[j],
                    out_hbm.at[pl.ds(bbase + c * _BC, _BC)],
                    osems[j],
                )
                @pl.when(c + 2 < NCH)
                def _():
                    fire_gather(c + 2, j)
            return carry

        lax.fori_loop(0, NCH // 2, body, 0)
        for j in range(2):
            pltpu.make_async_copy(
                outs[j], out_hbm.at[pl.ds(bbase, _BC)], osems[j]).wait()

    return emb


def kernel(token, lookup_table):
    BATCH = token.shape[0]
    B = BATCH * _T
    tab128 = _build_transpose()(lookup_table.astype(jnp.float32).T)
    idx = token.reshape(-1).astype(jnp.int32)
    idx = idx.reshape(_NW, (B // _NW) // _CHUNK, _CHUNK)
    return _build(BATCH)(idx, tab128)


# speedup vs baseline: 3.4010x; 1.0383x over previous
import functools
import math

import jax
import jax.numpy as jnp
from jax import lax
from jax.experimental import pallas as pl
from jax.experimental.pallas import tpu as pltpu
from jax.experimental.pallas import tpu_sc as plsc

_D = 64
_DP = 128
_SCALE = math.sqrt(_D)
_NC = 2
_NS = 16
_L = 16
_NW = _NC * _NS
_CHUNK = 80
_T = 20
_BC = _CHUNK // _T

_V = 1000000
_VBLK = 16384


@functools.lru_cache(maxsize=None)
def _build_transpose():
    grid = (_V + _VBLK - 1) // _VBLK

    def tr(tabt_ref, out_ref):
        out_ref[:, 0:_D] = tabt_ref[...].T

    return pl.pallas_call(
        tr,
        grid=(grid,),
        in_specs=[pl.BlockSpec((_D, _VBLK), lambda i: (0, i))],
        out_specs=pl.BlockSpec((_VBLK, _DP), lambda i: (i, 0)),
        out_shape=jax.ShapeDtypeStruct((_V, _DP), jnp.float32),
    )


@functools.lru_cache(maxsize=None)
def _build(BATCH: int):
    B = BATCH * _T
    BPW = B // _NW
    NCH = BPW // _CHUNK
    BW = BATCH // _NW

    mesh = plsc.VectorSubcoreMesh(core_axis_name="c", subcore_axis_name="s")

    @functools.partial(
        pl.kernel,
        mesh=mesh,
        out_type=jax.ShapeDtypeStruct((BATCH, _T, _D), jnp.float32),
        compiler_params=pltpu.CompilerParams(use_tc_tiling_on_sc=True),
        scratch_types=[
            pltpu.VMEM((NCH, _CHUNK), jnp.int32),
            pltpu.VMEM((_CHUNK, _DP), jnp.float32),
            pltpu.VMEM((_CHUNK, _DP), jnp.float32),
            pltpu.VMEM((_BC, _T, _D), jnp.float32),
            pltpu.VMEM((_BC, _T, _D), jnp.float32),
            pltpu.SemaphoreType.DMA,
            pltpu.SemaphoreType.DMA,
            pltpu.SemaphoreType.DMA,
            pltpu.SemaphoreType.DMA,
        ],
    )
    def emb(idx_hbm, tab_hbm, out_hbm, idx_v, ina, inb, outa, outb,
            gsa, gsb, osa, osb):
        wid = lax.axis_index("s") * _NC + lax.axis_index("c")
        bbase = wid * BW
        pltpu.sync_copy(idx_hbm.at[wid], idx_v)

        ins = (ina, inb)
        outs = (outa, outb)
        gsems = (gsa, gsb)
        osems = (osa, osb)

        def fire_gather(c, j):
            pltpu.async_copy(tab_hbm.at[idx_v.at[c]], ins[j], gsems[j])

        def compute(j):
            for r in range(_CHUNK):
                bl, tt = divmod(r, _T)
                for jj in range(_D // _L):
                    sl = pl.ds(jj * _L, _L)
                    outs[j][bl, tt, sl] = ins[j][r, sl] * _SCALE

        fire_gather(0, 0)
        fire_gather(1, 1)

        def body(k, carry):
            for j in range(2):
                c = 2 * k + j
                pltpu.make_async_copy(
                    tab_hbm.at[idx_v.at[0]], ins[j], gsems[j]).wait()
                @pl.when(k > 0)
                def _():
                    pltpu.make_async_copy(
                        outs[j], out_hbm.at[pl.ds(bbase, _BC)], osems[j]
                    ).wait()
                compute(j)
                pltpu.async_copy(
                    outs[j],
                    out_hbm.at[pl.ds(bbase + c * _BC, _BC)],
                    osems[j],
                )
                @pl.when(c + 2 < NCH)
                def _():
                    fire_gather(c + 2, j)
            return carry

        lax.fori_loop(0, NCH // 2, body, 0)
        for j in range(2):
            pltpu.make_async_copy(
                outs[j], out_hbm.at[pl.ds(bbase, _BC)], osems[j]).wait()

    return emb


def kernel(token, lookup_table):
    BATCH = token.shape[0]
    B = BATCH * _T
    tab128 = _build_transpose()(lookup_table.astype(jnp.float32).T)
    idx = token.reshape(-1).astype(jnp.int32)
    idx = idx.reshape(_NW, (B // _NW) // _CHUNK, _CHUNK)
    return _build(BATCH)(idx, tab128)


# speedup vs baseline: 3.4400x; 1.0115x over previous
import functools
import math

import jax
import jax.numpy as jnp
from jax import lax
from jax.experimental import pallas as pl
from jax.experimental.pallas import tpu as pltpu
from jax.experimental.pallas import tpu_sc as plsc

_D = 64
_DP = 128
_SCALE = math.sqrt(_D)
_NC = 2
_NS = 16
_L = 16
_NW = _NC * _NS
_CHUNK = 80
_T = 20
_BC = _CHUNK // _T

_V = 1000000
_VBLK = 32768


@functools.lru_cache(maxsize=None)
def _build_transpose():
    grid = (_V + _VBLK - 1) // _VBLK

    def tr(tabt_ref, out_ref):
        out_ref[:, 0:_D] = tabt_ref[...].T

    return pl.pallas_call(
        tr,
        grid=(grid,),
        in_specs=[pl.BlockSpec((_D, _VBLK), lambda i: (0, i))],
        out_specs=pl.BlockSpec((_VBLK, _DP), lambda i: (i, 0)),
        out_shape=jax.ShapeDtypeStruct((_V, _DP), jnp.float32),
    )


@functools.lru_cache(maxsize=None)
def _build(BATCH: int):
    B = BATCH * _T
    BPW = B // _NW
    NCH = BPW // _CHUNK
    BW = BATCH // _NW

    mesh = plsc.VectorSubcoreMesh(core_axis_name="c", subcore_axis_name="s")

    @functools.partial(
        pl.kernel,
        mesh=mesh,
        out_type=jax.ShapeDtypeStruct((BATCH, _T, _D), jnp.float32),
        compiler_params=pltpu.CompilerParams(use_tc_tiling_on_sc=True),
        scratch_types=[
            pltpu.VMEM((NCH, _CHUNK), jnp.int32),
            pltpu.VMEM((_CHUNK, _DP), jnp.float32),
            pltpu.VMEM((_CHUNK, _DP), jnp.float32),
            pltpu.VMEM((_BC, _T, _D), jnp.float32),
            pltpu.VMEM((_BC, _T, _D), jnp.float32),
            pltpu.SemaphoreType.DMA,
            pltpu.SemaphoreType.DMA,
            pltpu.SemaphoreType.DMA,
            pltpu.SemaphoreType.DMA,
        ],
    )
    def emb(idx_hbm, tab_hbm, out_hbm, idx_v, ina, inb, outa, outb,
            gsa, gsb, osa, osb):
        wid = lax.axis_index("s") * _NC + lax.axis_index("c")
        bbase = wid * BW
        pltpu.sync_copy(idx_hbm.at[wid], idx_v)

        ins = (ina, inb)
        outs = (outa, outb)
        gsems = (gsa, gsb)
        osems = (osa, osb)

        def fire_gather(c, j):
            pltpu.async_copy(tab_hbm.at[idx_v.at[c]], ins[j], gsems[j])

        def compute(j):
            for r in range(_CHUNK):
                bl, tt = divmod(r, _T)
                for jj in range(_D // _L):
                    sl = pl.ds(jj * _L, _L)
                    outs[j][bl, tt, sl] = ins[j][r, sl] * _SCALE

        fire_gather(0, 0)
        fire_gather(1, 1)

        def body(k, carry):
            for j in range(2):
                c = 2 * k + j
                pltpu.make_async_copy(
                    tab_hbm.at[idx_v.at[0]], ins[j], gsems[j]).wait()
                @pl.when(k > 0)
                def _():
                    pltpu.make_async_copy(
                        outs[j], out_hbm.at[pl.ds(bbase, _BC)], osems[j]
                    ).wait()
                compute(j)
                pltpu.async_copy(
                    outs[j],
                    out_hbm.at[pl.ds(bbase + c * _BC, _BC)],
                    osems[j],
                )
                @pl.when(c + 2 < NCH)
                def _():
                    fire_gather(c + 2, j)
            return carry

        lax.fori_loop(0, NCH // 2, body, 0)
        for j in range(2):
            pltpu.make_async_copy(
                outs[j], out_hbm.at[pl.ds(bbase, _BC)], osems[j]).wait()

    return emb


def kernel(token, lookup_table):
    BATCH = token.shape[0]
    B = BATCH * _T
    tab128 = _build_transpose()(lookup_table.astype(jnp.float32).T)
    idx = token.reshape(-1).astype(jnp.int32)
    idx = idx.reshape(_NW, (B // _NW) // _CHUNK, _CHUNK)
    return _build(BATCH)(idx, tab128)
